# baseline (speedup 1.0000x reference)
import functools

import jax
import jax.numpy as jnp
from jax import lax
from jax.experimental import pallas as pl
from jax.experimental.pallas import tpu as pltpu
from jax.experimental.pallas import tpu_sc as plsc

D_MODEL = 768
N_EXPERTS = 64
BLOCK = 4096
LANES = 128
L = 16


def _tc_body(x_ref, w_ref, b_ref, scores_t_ref):
    x = x_ref[...]
    w = w_ref[...]
    lt = jax.lax.dot_general(
        w, x, (((1,), (1,)), ((), ())), preferred_element_type=jnp.float32
    )
    lt = lt + b_ref[...]
    m = jnp.max(lt, axis=0, keepdims=True)
    ea = jnp.exp(lt - m)
    s = jnp.sum(ea, axis=0, keepdims=True)
    scores_t_ref[...] = ea * (1.0 / s)


def _make_sc_topk(n_tokens):
    nc, ns = 2, 16
    nw = nc * ns
    tok_w = n_tokens // nw
    groups = tok_w // L
    rows = tok_w // LANES

    mesh = plsc.VectorSubcoreMesh(core_axis_name="c", subcore_axis_name="s")

    @functools.partial(
        pl.kernel,
        mesh=mesh,
        out_type=[
            jax.ShapeDtypeStruct((n_tokens // LANES, 2, LANES), jnp.float32),
            jax.ShapeDtypeStruct((n_tokens // LANES, 2, LANES), jnp.int32),
        ],
        scratch_types=[
            pltpu.VMEM((N_EXPERTS, tok_w), jnp.float32),
            pltpu.VMEM((rows, 2, LANES), jnp.float32),
            pltpu.VMEM((rows, 2, LANES), jnp.int32),
        ],
    )
    def sc_topk(scores_t_hbm, vals_hbm, idx_hbm, sc_v, vals_v, idx_v):
        wid = lax.axis_index("s") * nc + lax.axis_index("c")
        base = wid * tok_w
        pltpu.sync_copy(scores_t_hbm.at[:, pl.ds(base, tok_w)], sc_v)

        def group(g, carry):
            off = g * L
            m1 = sc_v[0, pl.ds(off, L)]
            i1 = jnp.zeros((L,), jnp.int32)
            m2 = jnp.full((L,), float("-inf"), jnp.float32)
            i2 = jnp.zeros((L,), jnp.int32)
            for e in range(1, N_EXPERTS):
                v = sc_v[e, pl.ds(off, L)]
                gt1 = v > m1
                gt2 = v > m2
                i2 = jnp.where(gt1, i1, jnp.where(gt2, e, i2))
                m2 = jnp.where(gt1, m1, jnp.where(gt2, v, m2))
                i1 = jnp.where(gt1, e, i1)
                m1 = jnp.where(gt1, v, m1)
            r = g // (LANES // L)
            lo = (g % (LANES // L)) * L
            vals_v[r, 0, pl.ds(lo, L)] = m1
            vals_v[r, 1, pl.ds(lo, L)] = m2
            idx_v[r, 0, pl.ds(lo, L)] = i1
            idx_v[r, 1, pl.ds(lo, L)] = i2
            return carry

        lax.fori_loop(0, groups, group, 0)
        pltpu.sync_copy(vals_v, vals_hbm.at[pl.ds(base // LANES, rows)])
        pltpu.sync_copy(idx_v, idx_hbm.at[pl.ds(base // LANES, rows)])

    return sc_topk


@jax.jit
def kernel(hidden_states, gate_weight, gate_bias):
    n_tokens = hidden_states.shape[0]
    grid = (n_tokens // BLOCK,)
    bias2d = gate_bias.reshape(N_EXPERTS, 1)
    scores_t = pl.pallas_call(
        _tc_body,
        grid=grid,
        in_specs=[
            pl.BlockSpec((BLOCK, D_MODEL), lambda i: (i, 0)),
            pl.BlockSpec((N_EXPERTS, D_MODEL), lambda i: (0, 0)),
            pl.BlockSpec((N_EXPERTS, 1), lambda i: (0, 0)),
        ],
        out_specs=pl.BlockSpec((N_EXPERTS, BLOCK), lambda i: (0, i)),
        out_shape=jax.ShapeDtypeStruct((N_EXPERTS, n_tokens), jnp.float32),
        compiler_params=pltpu.CompilerParams(
            dimension_semantics=("parallel",),
        ),
    )(hidden_states, gate_weight, bias2d)
    vals3, idx3 = _make_sc_topk(n_tokens)(scores_t)
    scores = scores_t.T
    vals = vals3.transpose(0, 2, 1).reshape(n_tokens, 2)
    idx = idx3.transpose(0, 2, 1).reshape(n_tokens, 2)
    return ((idx, vals), scores)


# speedup vs baseline: 1.5756x; 1.5756x over previous
import jax
import jax.numpy as jnp
from jax.experimental import pallas as pl
from jax.experimental.pallas import tpu as pltpu

D_MODEL = 768
N_EXPERTS = 64
BLOCK = 2048
LANES = 128

_NEG_INF = float("-inf")


def _merge(a, b):
    av1, ae1, av2, ae2 = a
    bv1, be1, bv2, be2 = b
    a_wins = (av1 > bv1) | ((av1 == bv1) & (ae1 < be1))
    v1 = jnp.where(a_wins, av1, bv1)
    e1 = jnp.where(a_wins, ae1, be1)
    lv = jnp.where(a_wins, bv1, av1)
    le = jnp.where(a_wins, be1, ae1)
    cv = jnp.where(a_wins, av2, bv2)
    ce = jnp.where(a_wins, ae2, be2)
    l_wins = (lv > cv) | ((lv == cv) & (le < ce))
    v2 = jnp.where(l_wins, lv, cv)
    e2 = jnp.where(l_wins, le, ce)
    return (v1, e1, v2, e2)


def _router_body(x_ref, w_ref, b_ref, scores_t_ref, vals3_ref, idx3_ref):
    x = x_ref[...]
    w = w_ref[...]
    lt = jax.lax.dot_general(
        w, x, (((1,), (1,)), ((), ())), preferred_element_type=jnp.float32
    )
    lt = lt + b_ref[...]

    m1 = lt[0:8, :]
    c1 = jnp.zeros(m1.shape, jnp.int32)
    m2 = jnp.full(m1.shape, _NEG_INF, jnp.float32)
    c2 = jnp.zeros(m1.shape, jnp.int32)
    for c in range(1, 8):
        v = lt[8 * c : 8 * c + 8, :]
        gt1 = v > m1
        gt2 = v > m2
        c2 = jnp.where(gt1, c1, jnp.where(gt2, c, c2))
        m2 = jnp.where(gt1, m1, jnp.where(gt2, v, m2))
        c1 = jnp.where(gt1, c, c1)
        m1 = jnp.where(gt1, v, m1)

    siota = jax.lax.broadcasted_iota(jnp.int32, m1.shape, 0)
    t = (m1, c1 * 8 + siota, m2, c2 * 8 + siota)
    for h in (4, 2, 1):
        t = _merge(
            tuple(u[0:h, :] for u in t),
            tuple(u[h : 2 * h, :] for u in t),
        )
    v1, e1, v2, e2 = t

    ea = jnp.exp(lt - v1)
    s = jnp.sum(ea, axis=0, keepdims=True)
    inv = 1.0 / s
    scores_t_ref[...] = ea * inv

    g = BLOCK // LANES
    val1 = inv.reshape(g, LANES)
    val2 = (jnp.exp(v2 - v1) * inv).reshape(g, LANES)
    i1 = e1.reshape(g, LANES)
    i2 = e2.reshape(g, LANES)
    vals3_ref[...] = jnp.stack([val1, val2], axis=1)
    idx3_ref[...] = jnp.stack([i1, i2], axis=1)


@jax.jit
def kernel(hidden_states, gate_weight, gate_bias):
    n_tokens = hidden_states.shape[0]
    grid = (n_tokens // BLOCK,)
    g = BLOCK // LANES
    bias2d = gate_bias.reshape(N_EXPERTS, 1)
    scores_t, vals3, idx3 = pl.pallas_call(
        _router_body,
        grid=grid,
        in_specs=[
            pl.BlockSpec((BLOCK, D_MODEL), lambda i: (i, 0)),
            pl.BlockSpec((N_EXPERTS, D_MODEL), lambda i: (0, 0)),
            pl.BlockSpec((N_EXPERTS, 1), lambda i: (0, 0)),
        ],
        out_specs=[
            pl.BlockSpec((N_EXPERTS, BLOCK), lambda i: (0, i)),
            pl.BlockSpec((g, 2, LANES), lambda i: (i, 0, 0)),
            pl.BlockSpec((g, 2, LANES), lambda i: (i, 0, 0)),
        ],
        out_shape=[
            jax.ShapeDtypeStruct((N_EXPERTS, n_tokens), jnp.float32),
            jax.ShapeDtypeStruct((n_tokens // LANES, 2, LANES), jnp.float32),
            jax.ShapeDtypeStruct((n_tokens // LANES, 2, LANES), jnp.int32),
        ],
        compiler_params=pltpu.CompilerParams(
            dimension_semantics=("parallel",),
        ),
    )(hidden_states, gate_weight, bias2d)
    scores = scores_t.T
    vals = vals3.transpose(0, 2, 1).reshape(n_tokens, 2)
    idx = idx3.transpose(0, 2, 1).reshape(n_tokens, 2)
    return ((idx, vals), scores)


# speedup vs baseline: 1.5824x; 1.0043x over previous
import jax
import jax.numpy as jnp
from jax.experimental import pallas as pl
from jax.experimental.pallas import tpu as pltpu

D_MODEL = 768
N_EXPERTS = 64
BLOCK = 8192
LANES = 128

_NEG_INF = float("-inf")


def _merge(a, b):
    av1, ae1, av2, ae2 = a
    bv1, be1, bv2, be2 = b
    a_wins = (av1 > bv1) | ((av1 == bv1) & (ae1 < be1))
    v1 = jnp.where(a_wins, av1, bv1)
    e1 = jnp.where(a_wins, ae1, be1)
    lv = jnp.where(a_wins, bv1, av1)
    le = jnp.where(a_wins, be1, ae1)
    cv = jnp.where(a_wins, av2, bv2)
    ce = jnp.where(a_wins, ae2, be2)
    l_wins = (lv > cv) | ((lv == cv) & (le < ce))
    v2 = jnp.where(l_wins, lv, cv)
    e2 = jnp.where(l_wins, le, ce)
    return (v1, e1, v2, e2)


def _router_body(x_ref, w_ref, b_ref, scores_t_ref, vals3_ref, idx3_ref):
    x = x_ref[...]
    w = w_ref[...]
    lt = jax.lax.dot_general(
        w, x, (((1,), (1,)), ((), ())), preferred_element_type=jnp.float32
    )
    lt = lt + b_ref[...]

    m1 = lt[0:8, :]
    c1 = jnp.zeros(m1.shape, jnp.int32)
    m2 = jnp.full(m1.shape, _NEG_INF, jnp.float32)
    c2 = jnp.zeros(m1.shape, jnp.int32)
    for c in range(1, 8):
        v = lt[8 * c : 8 * c + 8, :]
        gt1 = v > m1
        gt2 = v > m2
        c2 = jnp.where(gt1, c1, jnp.where(gt2, c, c2))
        m2 = jnp.where(gt1, m1, jnp.where(gt2, v, m2))
        c1 = jnp.where(gt1, c, c1)
        m1 = jnp.where(gt1, v, m1)

    siota = jax.lax.broadcasted_iota(jnp.int32, m1.shape, 0)
    t = (m1, c1 * 8 + siota, m2, c2 * 8 + siota)
    for h in (4, 2, 1):
        t = _merge(
            tuple(u[0:h, :] for u in t),
            tuple(u[h : 2 * h, :] for u in t),
        )
    v1, e1, v2, e2 = t

    ea = jnp.exp(lt - v1)
    s = jnp.sum(ea, axis=0, keepdims=True)
    inv = 1.0 / s
    scores_t_ref[...] = ea * inv

    g = BLOCK // LANES
    val1 = inv.reshape(g, LANES)
    val2 = (jnp.exp(v2 - v1) * inv).reshape(g, LANES)
    i1 = e1.reshape(g, LANES)
    i2 = e2.reshape(g, LANES)
    vals3_ref[...] = jnp.stack([val1, val2], axis=1)
    idx3_ref[...] = jnp.stack([i1, i2], axis=1)


@jax.jit
def kernel(hidden_states, gate_weight, gate_bias):
    n_tokens = hidden_states.shape[0]
    grid = (n_tokens // BLOCK,)
    g = BLOCK // LANES
    bias2d = gate_bias.reshape(N_EXPERTS, 1)
    scores_t, vals3, idx3 = pl.pallas_call(
        _router_body,
        grid=grid,
        in_specs=[
            pl.BlockSpec((BLOCK, D_MODEL), lambda i: (i, 0)),
            pl.BlockSpec((N_EXPERTS, D_MODEL), lambda i: (0, 0)),
            pl.BlockSpec((N_EXPERTS, 1), lambda i: (0, 0)),
        ],
        out_specs=[
            pl.BlockSpec((N_EXPERTS, BLOCK), lambda i: (0, i)),
            pl.BlockSpec((g, 2, LANES), lambda i: (i, 0, 0)),
            pl.BlockSpec((g, 2, LANES), lambda i: (i, 0, 0)),
        ],
        out_shape=[
            jax.ShapeDtypeStruct((N_EXPERTS, n_tokens), jnp.float32),
            jax.ShapeDtypeStruct((n_tokens // LANES, 2, LANES), jnp.float32),
            jax.ShapeDtypeStruct((n_tokens // LANES, 2, LANES), jnp.int32),
        ],
        compiler_params=pltpu.CompilerParams(
            dimension_semantics=("parallel",),
        ),
    )(hidden_states, gate_weight, bias2d)
    scores = scores_t.T
    vals = vals3.transpose(0, 2, 1).reshape(n_tokens, 2)
    idx = idx3.transpose(0, 2, 1).reshape(n_tokens, 2)
    return ((idx, vals), scores)


# speedup vs baseline: 1.6949x; 1.0711x over previous
import jax
import jax.numpy as jnp
from jax.experimental import pallas as pl
from jax.experimental.pallas import tpu as pltpu

D_MODEL = 768
N_EXPERTS = 64
BLOCK = 4096
LANES = 128

_NEG_INF = float("-inf")


def _merge(a, b):
    av1, ae1, av2, ae2 = a
    bv1, be1, bv2, be2 = b
    a_wins = (av1 > bv1) | ((av1 == bv1) & (ae1 < be1))
    v1 = jnp.where(a_wins, av1, bv1)
    e1 = jnp.where(a_wins, ae1, be1)
    lv = jnp.where(a_wins, bv1, av1)
    le = jnp.where(a_wins, be1, ae1)
    cv = jnp.where(a_wins, av2, bv2)
    ce = jnp.where(a_wins, ae2, be2)
    l_wins = (lv > cv) | ((lv == cv) & (le < ce))
    v2 = jnp.where(l_wins, lv, cv)
    e2 = jnp.where(l_wins, le, ce)
    return (v1, e1, v2, e2)


def _router_body(x_ref, w_ref, b_ref, scores_t_ref, vals3_ref, idx3_ref):
    x = x_ref[...]
    w = w_ref[...]
    lt = jax.lax.dot_general(
        w, x, (((1,), (1,)), ((), ())), preferred_element_type=jnp.float32
    )
    lt = lt + b_ref[...]

    m1 = lt[0:8, :]
    c1 = jnp.zeros(m1.shape, jnp.int32)
    m2 = jnp.full(m1.shape, _NEG_INF, jnp.float32)
    c2 = jnp.zeros(m1.shape, jnp.int32)
    for c in range(1, 8):
        v = lt[8 * c : 8 * c + 8, :]
        gt1 = v > m1
        gt2 = v > m2
        c2 = jnp.where(gt1, c1, jnp.where(gt2, c, c2))
        m2 = jnp.where(gt1, m1, jnp.where(gt2, v, m2))
        c1 = jnp.where(gt1, c, c1)
        m1 = jnp.where(gt1, v, m1)

    siota = jax.lax.broadcasted_iota(jnp.int32, m1.shape, 0)
    t = (m1, c1 * 8 + siota, m2, c2 * 8 + siota)
    for h in (4, 2, 1):
        t = _merge(
            tuple(u[0:h, :] for u in t),
            tuple(u[h : 2 * h, :] for u in t),
        )
    v1, e1, v2, e2 = t

    ea = jnp.exp(lt - v1)
    s = jnp.sum(ea, axis=0, keepdims=True)
    inv = 1.0 / s
    scores_t_ref[...] = ea * inv

    g = BLOCK // LANES
    val1 = inv.reshape(g, LANES)
    val2 = (jnp.exp(v2 - v1) * inv).reshape(g, LANES)
    i1 = e1.reshape(g, LANES)
    i2 = e2.reshape(g, LANES)
    vals3_ref[...] = jnp.stack([val1, val2], axis=1)
    idx3_ref[...] = jnp.stack([i1, i2], axis=1)


@jax.jit
def kernel(hidden_states, gate_weight, gate_bias):
    n_tokens = hidden_states.shape[0]
    grid = (n_tokens // BLOCK,)
    g = BLOCK // LANES
    bias2d = gate_bias.reshape(N_EXPERTS, 1)
    scores_t, vals3, idx3 = pl.pallas_call(
        _router_body,
        grid=grid,
        in_specs=[
            pl.BlockSpec((BLOCK, D_MODEL), lambda i: (i, 0)),
            pl.BlockSpec((N_EXPERTS, D_MODEL), lambda i: (0, 0)),
            pl.BlockSpec((N_EXPERTS, 1), lambda i: (0, 0)),
        ],
        out_specs=[
            pl.BlockSpec((N_EXPERTS, BLOCK), lambda i: (0, i)),
            pl.BlockSpec((g, 2, LANES), lambda i: (i, 0, 0)),
            pl.BlockSpec((g, 2, LANES), lambda i: (i, 0, 0)),
        ],
        out_shape=[
            jax.ShapeDtypeStruct((N_EXPERTS, n_tokens), jnp.float32),
            jax.ShapeDtypeStruct((n_tokens // LANES, 2, LANES), jnp.float32),
            jax.ShapeDtypeStruct((n_tokens // LANES, 2, LANES), jnp.int32),
        ],
        compiler_params=pltpu.CompilerParams(
            dimension_semantics=("parallel",),
        ),
    )(hidden_states, gate_weight, bias2d)
    scores = scores_t.T
    vals = vals3.transpose(0, 2, 1).reshape(n_tokens, 2)
    idx = idx3.transpose(0, 2, 1).reshape(n_tokens, 2)
    return ((idx, vals), scores)


# speedup vs baseline: 1.7598x; 1.0383x over previous
import jax
import jax.numpy as jnp
from jax.experimental import pallas as pl
from jax.experimental.pallas import tpu as pltpu

D_MODEL = 768
N_EXPERTS = 64
BLOCK = 4096
LANES = 128

_NEG_INF = float("-inf")


def _merge(a, b):
    av1, ae1, av2, ae2 = a
    bv1, be1, bv2, be2 = b
    a_wins = (av1 > bv1) | ((av1 == bv1) & (ae1 < be1))
    v1 = jnp.where(a_wins, av1, bv1)
    e1 = jnp.where(a_wins, ae1, be1)
    lv = jnp.where(a_wins, bv1, av1)
    le = jnp.where(a_wins, be1, ae1)
    cv = jnp.where(a_wins, av2, bv2)
    ce = jnp.where(a_wins, ae2, be2)
    l_wins = (lv > cv) | ((lv == cv) & (le < ce))
    v2 = jnp.where(l_wins, lv, cv)
    e2 = jnp.where(l_wins, le, ce)
    return (v1, e1, v2, e2)


def _router_body(x_ref, w_ref, b_ref, scores_t_ref, vals3_ref, idx3_ref):
    x = x_ref[...]
    w = w_ref[...]
    lt = jax.lax.dot_general(
        w, x, (((1,), (1,)), ((), ())), preferred_element_type=jnp.float32
    )
    lt = lt + b_ref[...].reshape(N_EXPERTS, 1)

    m1 = lt[0:8, :]
    c1 = jnp.zeros(m1.shape, jnp.int32)
    m2 = jnp.full(m1.shape, _NEG_INF, jnp.float32)
    c2 = jnp.zeros(m1.shape, jnp.int32)
    for c in range(1, 8):
        v = lt[8 * c : 8 * c + 8, :]
        gt1 = v > m1
        gt2 = v > m2
        c2 = jnp.where(gt1, c1, jnp.where(gt2, c, c2))
        m2 = jnp.where(gt1, m1, jnp.where(gt2, v, m2))
        c1 = jnp.where(gt1, c, c1)
        m1 = jnp.where(gt1, v, m1)

    siota = jax.lax.broadcasted_iota(jnp.int32, m1.shape, 0)
    t = (m1, c1 * 8 + siota, m2, c2 * 8 + siota)
    for h in (4, 2, 1):
        t = _merge(
            tuple(u[0:h, :] for u in t),
            tuple(u[h : 2 * h, :] for u in t),
        )
    v1, e1, v2, e2 = t

    ea = jnp.exp(lt - v1)
    s = jnp.sum(ea, axis=0, keepdims=True)
    inv = 1.0 / s
    scores_t_ref[...] = ea * inv

    g = BLOCK // LANES
    val1 = inv.reshape(g, LANES)
    val2 = (jnp.exp(v2 - v1) * inv).reshape(g, LANES)
    i1 = e1.reshape(g, LANES)
    i2 = e2.reshape(g, LANES)
    vals3_ref[...] = jnp.stack([val1, val2], axis=1)
    idx3_ref[...] = jnp.stack([i1, i2], axis=1)


@jax.jit
def kernel(hidden_states, gate_weight, gate_bias):
    n_tokens = hidden_states.shape[0]
    grid = (n_tokens // BLOCK,)
    g = BLOCK // LANES
    bias2d = gate_bias.reshape(1, N_EXPERTS)
    scores_t, vals3, idx3 = pl.pallas_call(
        _router_body,
        grid=grid,
        in_specs=[
            pl.BlockSpec((BLOCK, D_MODEL), lambda i: (i, 0)),
            pl.BlockSpec((N_EXPERTS, D_MODEL), lambda i: (0, 0)),
            pl.BlockSpec((1, N_EXPERTS), lambda i: (0, 0)),
        ],
        out_specs=[
            pl.BlockSpec((N_EXPERTS, BLOCK), lambda i: (0, i)),
            pl.BlockSpec((g, 2, LANES), lambda i: (i, 0, 0)),
            pl.BlockSpec((g, 2, LANES), lambda i: (i, 0, 0)),
        ],
        out_shape=[
            jax.ShapeDtypeStruct((N_EXPERTS, n_tokens), jnp.float32),
            jax.ShapeDtypeStruct((n_tokens // LANES, 2, LANES), jnp.float32),
            jax.ShapeDtypeStruct((n_tokens // LANES, 2, LANES), jnp.int32),
        ],
        compiler_params=pltpu.CompilerParams(
            dimension_semantics=("parallel",),
        ),
    )(hidden_states, gate_weight, bias2d)
    scores = scores_t.T
    vals = vals3.transpose(0, 2, 1).reshape(n_tokens, 2)
    idx = idx3.transpose(0, 2, 1).reshape(n_tokens, 2)
    return ((idx, vals), scores)
